# Initial kernel scaffold; baseline (speedup 1.0000x reference)
import functools

import jax
import jax.numpy as jnp
from jax import lax
from jax.experimental import pallas as pl
from jax.experimental.pallas import tpu as pltpu
from jax.experimental.pallas import tpu_sc as plsc

N = 50000
E = 320000
D = 128
H = 16
NC = 2
NS = 16
NW = NC * NS
CH = 128
SCH = 1024
S = 10
EPW = S * SCH
EP = NW * EPW
K = EPW // CH
NPAD = 51200
DUMP = NPAD - 128
RPS = NPAD // NS
ZR = 800

_MESH = plsc.VectorSubcoreMesh(
    core_axis_name="c", subcore_axis_name="s", num_cores=NC, num_subcores=NS)

_f32 = jnp.float32


@functools.partial(
    pl.kernel,
    out_type=(jax.ShapeDtypeStruct((NW, S, SCH, H), _f32),
              jax.ShapeDtypeStruct((NW, S, SCH, H), _f32)),
    mesh=_MESH,
    scratch_types=(
        pltpu.VMEM((K, CH), jnp.int32),
        pltpu.VMEM((K, CH), jnp.int32),
        pltpu.VMEM((2, SCH, H), _f32),
        pltpu.VMEM((2, SCH, H), _f32),
        pltpu.SemaphoreType.DMA,
        pltpu.SemaphoreType.DMA,
        pltpu.SemaphoreType.DMA,
        pltpu.SemaphoreType.DMA,
    ),
)
def _sc_gather(t1_hbm, t2_hbm, idxt_hbm, idxs_hbm, out1_hbm, out2_hbm,
               idxt_v, idxs_v, bufA, bufB, semg0, semg1, semw0, semw1):
    w = lax.axis_index("s") * NC + lax.axis_index("c")
    pltpu.sync_copy(idxt_hbm.at[w], idxt_v)
    pltpu.sync_copy(idxs_hbm.at[w], idxs_v)
    semg = (semg0, semg1)
    semw = (semw0, semw1)
    gh = {}
    wh = {}
    for s in range(S + 1):
        b = s % 2
        if s < S:
            for hnd in wh.pop(b, ()):
                hnd.wait()
            g = []
            for j in range(8):
                k = s * 8 + j
                dst = pl.ds(j * CH, CH)
                g.append(pltpu.async_copy(
                    t1_hbm.at[idxt_v.at[k]], bufA.at[b, dst], semg[b]))
                g.append(pltpu.async_copy(
                    t2_hbm.at[idxs_v.at[k]], bufB.at[b, dst], semg[b]))
            gh[b] = g
        if s >= 1:
            pb = (s - 1) % 2
            for hnd in gh.pop(pb):
                hnd.wait()
            wh[pb] = [
                pltpu.async_copy(bufA.at[pb], out1_hbm.at[w, s - 1], semw[pb]),
                pltpu.async_copy(bufB.at[pb], out2_hbm.at[w, s - 1], semw[pb]),
            ]
    for b in (0, 1):
        for hnd in wh.pop(b, ()):
            hnd.wait()


@functools.partial(
    pl.kernel,
    out_type=(jax.ShapeDtypeStruct((NC, NPAD, H), _f32),
              jax.ShapeDtypeStruct((NC, NPAD, H), _f32)),
    mesh=_MESH,
    scratch_types=(
        pltpu.VMEM_SHARED((NPAD, H), _f32),
        pltpu.VMEM_SHARED((NPAD, H), _f32),
        pltpu.VMEM((ZR, H), _f32),
        pltpu.VMEM((CH, H), _f32),
        pltpu.VMEM((K, CH), jnp.int32),
        pltpu.VMEM((2, SCH, H), _f32),
        pltpu.SemaphoreType.DMA,
        pltpu.SemaphoreType.DMA,
        pltpu.SemaphoreType.DMA,
        pltpu.SemaphoreType.DMA,
        pltpu.SemaphoreType.DMA,
    ),
)
def _sc_scatter(y_hbm, idx_hbm, sum_hbm, cnt_hbm,
                acc_v, acc_c, zbuf, obuf, idx_v, ybuf,
                semz, seml0, seml1, sems0, sems1):
    c = lax.axis_index("c")
    sid = lax.axis_index("s")
    w = sid * NC + c

    def fill_z(i, carry):
        zbuf[i] = jnp.zeros((H,), _f32)
        return carry

    def fill_o(i, carry):
        obuf[i] = jnp.full((H,), 1.0, _f32)
        return carry

    lax.fori_loop(0, ZR, fill_z, 0, unroll=8)
    lax.fori_loop(0, CH, fill_o, 0, unroll=8)

    zh = []
    for q in range(4):
        row = pl.ds(sid * RPS + q * ZR, ZR)
        zh.append(pltpu.async_copy(zbuf, acc_v.at[row], semz))
        zh.append(pltpu.async_copy(zbuf, acc_c.at[row], semz))
    for hnd in zh:
        hnd.wait()
    plsc.subcore_barrier()

    pltpu.sync_copy(idx_hbm.at[w], idx_v)
    seml = (seml0, seml1)
    sems = (sems0, sems1)
    lh = {}
    sh = {}
    for s in range(S + 1):
        b = s % 2
        if s < S:
            for hnd in sh.pop(b, ()):
                hnd.wait()
            lh[b] = pltpu.async_copy(y_hbm.at[w, s], ybuf.at[b], seml[b])
        if s >= 1:
            pb = (s - 1) % 2
            lh.pop(pb).wait()
            g = []
            for j in range(8):
                k = (s - 1) * 8 + j
                g.append(pltpu.async_copy(
                    ybuf.at[pb, pl.ds(j * CH, CH)], acc_v.at[idx_v.at[k]],
                    sems[pb], add=True))
                g.append(pltpu.async_copy(
                    obuf, acc_c.at[idx_v.at[k]], sems[pb], add=True))
            sh[pb] = g
    for b in (0, 1):
        for hnd in sh.pop(b, ()):
            hnd.wait()
    plsc.subcore_barrier()

    row = pl.ds(sid * RPS, RPS)
    pltpu.sync_copy(acc_v.at[row], sum_hbm.at[c, row])
    pltpu.sync_copy(acc_c.at[row], cnt_hbm.at[c, row])


_BN = 1000
_BE = 2048


def _dot(a, b):
    return jnp.dot(a, b, preferred_element_type=_f32)


def _proj_body(xv, xc, xa, wv, wc, wa,
               o_gvv, o_hvv, o_fvx, o_gvc, o_hva, o_gaa, o_fax):
    pv = _dot(xv[...], wv[...])
    o_gvv[...] = pv[:, 0:16]
    o_hvv[...] = pv[:, 16:32]
    o_fvx[...] = pv[:, 32:48]
    o_gvc[...] = _dot(xc[...], wc[...])
    pa = _dot(xa[...], wa[...])
    o_hva[...] = pa[:, 0:16]
    o_gaa[...] = pa[:, 16:32]
    o_fax[...] = pa[:, 32:48]


def _edge_body(gt, gs, ea8, we8, w2t, b2r, out):
    z = gt[...] + gs[...] + _dot(ea8[...], we8[...])
    u = jnp.maximum(z, 0.0)
    out[...] = jnp.maximum(_dot(u, w2t[...]) + b2r[...][0:1, :], 0.0)


def _fv_body(sg, cg, sh, ch, pfx, wg, wh, b1r, w2t, b2r, wf, out):
    sg_, cg_, sh_, ch_ = sg[...], cg[...], sh[...], ch[...]
    ag = (sg_[0] + sg_[1]) / jnp.maximum(cg_[0] + cg_[1], 1.0)
    ah = (sh_[0] + sh_[1]) / jnp.maximum(ch_[0] + ch_[1], 1.0)
    z = pfx[...] + _dot(ag, wg[...]) + _dot(ah, wh[...]) + b1r[...][0:1, :]
    u = jnp.maximum(z, 0.0)
    fv = jnp.maximum(_dot(u, w2t[...]) + b2r[...][0:1, :], 0.0)
    out[...] = _dot(fv, wf[...])


def _fa_body(sa, ca, pfx, wg, b1r, w2t, b2r, out):
    sa_, ca_ = sa[...], ca[...]
    ag = (sa_[0] + sa_[1]) / jnp.maximum(ca_[0] + ca_[1], 1.0)
    z = pfx[...] + _dot(ag, wg[...]) + b1r[...][0:1, :]
    u = jnp.maximum(z, 0.0)
    out[...] = jnp.maximum(_dot(u, w2t[...]) + b2r[...][0:1, :], 0.0)


def _full(shape):
    return pl.BlockSpec(shape, lambda i: tuple(0 for _ in shape))


def _rows(bn, cols):
    return pl.BlockSpec((bn, cols), lambda i: (i, 0))


def _rows3(bn, cols):
    return pl.BlockSpec((NC, bn, cols), lambda i: (0, i, 0))


_n16 = jax.ShapeDtypeStruct((N, H), _f32)

_proj_call = pl.pallas_call(
    _proj_body,
    grid=(N // _BN,),
    in_specs=[_rows(_BN, D)] * 3 + [_full((D, 48)), _full((D, 16)),
                                    _full((D, 48))],
    out_specs=[_rows(_BN, H)] * 7,
    out_shape=tuple([_n16] * 7),
)

_edge_call = pl.pallas_call(
    _edge_body,
    grid=(EP // _BE,),
    in_specs=[_rows(_BE, H), _rows(_BE, H), _rows(_BE, 8),
              _full((8, H)), _full((H, H)), _full((8, H))],
    out_specs=_rows(_BE, H),
    out_shape=jax.ShapeDtypeStruct((EP, H), _f32),
)

_fv_call = pl.pallas_call(
    _fv_body,
    grid=(N // _BN,),
    in_specs=[_rows3(_BN, H)] * 4 + [_rows(_BN, H)] +
             [_full((H, H)), _full((H, H)), _full((8, H)), _full((H, H)),
              _full((8, H)), _full((H, H))],
    out_specs=_rows(_BN, H),
    out_shape=_n16,
)

_fa_call = pl.pallas_call(
    _fa_body,
    grid=(N // _BN,),
    in_specs=[_rows3(_BN, H)] * 2 + [_rows(_BN, H)] +
             [_full((H, H)), _full((8, H)), _full((H, H)), _full((8, H))],
    out_specs=_rows(_BN, H),
    out_shape=_n16,
)


def _pad_idx(idx, fill):
    p = jnp.pad(idx, (0, EP - E), constant_values=fill)
    return p.reshape(NW, K, CH)


def _rep8(b):
    return jnp.tile(b[None, :], (8, 1))


def kernel(x_c, x_v, x_a, edge_index_c2v, edge_index_a2v,
           edge_attr_c2v, edge_attr_a2v,
           gv_W1, gv_b1, gv_W2, gv_b2,
           hv_W1, hv_b1, hv_W2, hv_b2,
           fv_W1, fv_b1, fv_W2, fv_b2,
           ga_W1, ga_b1, ga_W2, ga_b2,
           fa_W1, fa_b1, fa_W2, fa_b2):
    c2v_s, c2v_t = edge_index_c2v[0], edge_index_c2v[1]
    a2v_s, a2v_t = edge_index_a2v[0], edge_index_a2v[1]

    wv = jnp.concatenate([gv_W1[:, :D], hv_W1[:, :D], fv_W1[:, :D]], 0).T
    wc = gv_W1[:, D:2 * D].T
    wa = jnp.concatenate([hv_W1[:, D:2 * D], ga_W1[:, :D], fa_W1[:, :D]], 0).T

    p_gvv, p_hvv, p_fvx, p_gvc, p_hva, p_gaa, p_fax = _proj_call(
        x_v, x_c, x_a, wv, wc, wa)

    def ea8(ea, wcols, b1):
        e8 = jnp.pad(jnp.concatenate([ea, jnp.ones((E, 1), _f32)], 1),
                     ((0, EP - E), (0, 3)))
        w8 = jnp.pad(jnp.concatenate([wcols, b1[:, None]], 1),
                     ((0, 0), (0, 3))).T
        return e8, w8

    eac8, wec = ea8(edge_attr_c2v, gv_W1[:, 2 * D:], gv_b1)
    eaa8, weh = ea8(edge_attr_a2v, hv_W1[:, 2 * D:], hv_b1)
    _, wea = ea8(edge_attr_a2v, ga_W1[:, D + H:], ga_b1)

    c2v_t_g = _pad_idx(c2v_t, 0)
    c2v_s_g = _pad_idx(c2v_s, 0)
    c2v_t_s = _pad_idx(c2v_t, DUMP)
    a2v_t_g = _pad_idx(a2v_t, 0)
    a2v_s_g = _pad_idx(a2v_s, 0)
    a2v_t_s = _pad_idx(a2v_t, DUMP)
    a2v_s_s = _pad_idx(a2v_s, DUMP)

    def as_edges(x):
        return x.reshape(EP, H)

    def as_chunks(x):
        return x.reshape(NW, S, SCH, H)

    g_t, g_s = _sc_gather(p_gvv, p_gvc, c2v_t_g, c2v_s_g)
    y_gv = _edge_call(as_edges(g_t), as_edges(g_s), eac8, wec, gv_W2.T,
                      _rep8(gv_b2))
    s_g, c_g = _sc_scatter(as_chunks(y_gv), c2v_t_s)

    h_t, h_s = _sc_gather(p_hvv, p_hva, a2v_t_g, a2v_s_g)
    y_hv = _edge_call(as_edges(h_t), as_edges(h_s), eaa8, weh, hv_W2.T,
                      _rep8(hv_b2))
    s_h, c_h = _sc_scatter(as_chunks(y_hv), a2v_t_s)

    p_gaf = _fv_call(s_g, c_g, s_h, c_h, p_fvx,
                     fv_W1[:, D:D + H].T, fv_W1[:, D + H:].T, _rep8(fv_b1),
                     fv_W2.T, _rep8(fv_b2), ga_W1[:, D:D + H].T)

    f_t, f_s = _sc_gather(p_gaf, p_gaa, a2v_t_g, a2v_s_g)
    y_ga = _edge_call(as_edges(f_t), as_edges(f_s), eaa8, wea, ga_W2.T,
                      _rep8(ga_b2))
    s_a, c_a = _sc_scatter(as_chunks(y_ga), a2v_s_s)

    return _fa_call(s_a, c_a, p_fax,
                    fa_W1[:, D:].T, _rep8(fa_b1), fa_W2.T, _rep8(fa_b2))


# speedup vs baseline: 2.2106x; 2.2106x over previous
import functools

import jax
import jax.numpy as jnp
from jax import lax
from jax.experimental import pallas as pl
from jax.experimental.pallas import tpu as pltpu
from jax.experimental.pallas import tpu_sc as plsc

N = 50000
E = 320000
D = 128
H = 16
NC = 2
NS = 16
NW = NC * NS
CH = 128
SCH = 1024
S = 10
EPW = S * SCH
EP = NW * EPW
K = EPW // CH
NPAD = 50176
DUMP = NPAD - 128
RPS = NPAD // NS
SCHS = 512
SS = EPW // SCHS

_MESH = plsc.VectorSubcoreMesh(
    core_axis_name="c", subcore_axis_name="s", num_cores=NC, num_subcores=NS)

_SC_PARAMS = pltpu.CompilerParams(use_tc_tiling_on_sc=False)

_f32 = jnp.float32


@functools.partial(
    pl.kernel,
    out_type=(jax.ShapeDtypeStruct((NW, S, SCH, H), _f32),
              jax.ShapeDtypeStruct((NW, S, SCH, H), _f32)),
    mesh=_MESH,
    scratch_types=(
        pltpu.VMEM((K, CH), jnp.int32),
        pltpu.VMEM((K, CH), jnp.int32),
        pltpu.VMEM((2, SCH, H), _f32),
        pltpu.VMEM((2, SCH, H), _f32),
        pltpu.SemaphoreType.DMA,
        pltpu.SemaphoreType.DMA,
        pltpu.SemaphoreType.DMA,
        pltpu.SemaphoreType.DMA,
    ),
    compiler_params=_SC_PARAMS,
)
def _sc_gather(t1_hbm, t2_hbm, idxt_hbm, idxs_hbm, out1_hbm, out2_hbm,
               idxt_v, idxs_v, bufA, bufB, semg0, semg1, semw0, semw1):
    w = lax.axis_index("s") * NC + lax.axis_index("c")
    pltpu.sync_copy(idxt_hbm.at[w], idxt_v)
    pltpu.sync_copy(idxs_hbm.at[w], idxs_v)
    semg = (semg0, semg1)
    semw = (semw0, semw1)
    gh = {}
    wh = {}
    for s in range(S + 1):
        b = s % 2
        if s < S:
            for hnd in wh.pop(b, ()):
                hnd.wait()
            g = []
            for j in range(8):
                k = s * 8 + j
                dst = pl.ds(j * CH, CH)
                g.append(pltpu.async_copy(
                    t1_hbm.at[idxt_v.at[k]], bufA.at[b, dst], semg[b]))
                g.append(pltpu.async_copy(
                    t2_hbm.at[idxs_v.at[k]], bufB.at[b, dst], semg[b]))
            gh[b] = g
        if s >= 1:
            pb = (s - 1) % 2
            for hnd in gh.pop(pb):
                hnd.wait()
            wh[pb] = [
                pltpu.async_copy(bufA.at[pb], out1_hbm.at[w, s - 1], semw[pb]),
                pltpu.async_copy(bufB.at[pb], out2_hbm.at[w, s - 1], semw[pb]),
            ]
    for b in (0, 1):
        for hnd in wh.pop(b, ()):
            hnd.wait()


@functools.partial(
    pl.kernel,
    out_type=(jax.ShapeDtypeStruct((NC, NPAD, H), _f32),
              jax.ShapeDtypeStruct((NC, NPAD, H), _f32)),
    mesh=_MESH,
    scratch_types=(
        pltpu.VMEM_SHARED((NPAD, H), _f32),
        pltpu.VMEM_SHARED((NPAD, H), _f32),
        pltpu.VMEM((CH, H), _f32),
        pltpu.VMEM((K, CH), jnp.int32),
        pltpu.VMEM((2, SCHS, H), _f32),
        pltpu.SemaphoreType.DMA,
        pltpu.SemaphoreType.DMA,
        pltpu.SemaphoreType.DMA,
        pltpu.SemaphoreType.DMA,
        pltpu.SemaphoreType.DMA,
    ),
    compiler_params=_SC_PARAMS,
)
def _sc_scatter(y_hbm, idx_hbm, zeros_hbm, sum_hbm, cnt_hbm,
                acc_v, acc_c, obuf, idx_v, ybuf,
                semz, seml0, seml1, sems0, sems1):
    c = lax.axis_index("c")
    sid = lax.axis_index("s")
    w = sid * NC + c

    def fill_o(i, carry):
        obuf[i] = jnp.full((H,), 1.0, _f32)
        return carry

    lax.fori_loop(0, CH, fill_o, 0, unroll=8)

    row = pl.ds(sid * RPS, RPS)
    z0 = pltpu.async_copy(zeros_hbm, acc_v.at[row], semz)
    z1 = pltpu.async_copy(zeros_hbm, acc_c.at[row], semz)
    z0.wait()
    z1.wait()
    plsc.subcore_barrier()

    pltpu.sync_copy(idx_hbm.at[w], idx_v)
    seml = (seml0, seml1)
    sems = (sems0, sems1)
    lh = {}
    sh = {}
    for s in range(SS + 1):
        b = s % 2
        if s < SS:
            for hnd in sh.pop(b, ()):
                hnd.wait()
            lh[b] = pltpu.async_copy(y_hbm.at[w, s], ybuf.at[b], seml[b])
        if s >= 1:
            pb = (s - 1) % 2
            lh.pop(pb).wait()
            g = []
            for j in range(SCHS // CH):
                k = (s - 1) * (SCHS // CH) + j
                g.append(pltpu.async_copy(
                    ybuf.at[pb, pl.ds(j * CH, CH)], acc_v.at[idx_v.at[k]],
                    sems[pb], add=True))
                g.append(pltpu.async_copy(
                    obuf, acc_c.at[idx_v.at[k]], sems[pb], add=True))
            sh[pb] = g
    for b in (0, 1):
        for hnd in sh.pop(b, ()):
            hnd.wait()
    plsc.subcore_barrier()

    row = pl.ds(sid * RPS, RPS)
    pltpu.sync_copy(acc_v.at[row], sum_hbm.at[c, row])
    pltpu.sync_copy(acc_c.at[row], cnt_hbm.at[c, row])


_BN = 1000
_BE = 2048


def _dot(a, b):
    return jnp.dot(a, b, preferred_element_type=_f32)


def _proj_body(xv, xc, xa, wv, wc, wa,
               o_gvv, o_hvv, o_fvx, o_gvc, o_hva, o_gaa, o_fax):
    pv = _dot(xv[...], wv[...])
    o_gvv[...] = pv[:, 0:16]
    o_hvv[...] = pv[:, 16:32]
    o_fvx[...] = pv[:, 32:48]
    o_gvc[...] = _dot(xc[...], wc[...])
    pa = _dot(xa[...], wa[...])
    o_hva[...] = pa[:, 0:16]
    o_gaa[...] = pa[:, 16:32]
    o_fax[...] = pa[:, 32:48]


def _edge_body(gt, gs, ea8, we8, w2t, b2r, out):
    z = gt[...] + gs[...] + _dot(ea8[...], we8[...])
    u = jnp.maximum(z, 0.0)
    out[...] = jnp.maximum(_dot(u, w2t[...]) + b2r[...][0:1, :], 0.0)


def _fv_body(sg, cg, sh, ch, pfx, wg, wh, b1r, w2t, b2r, wf, out):
    sg_, cg_, sh_, ch_ = sg[...], cg[...], sh[...], ch[...]
    ag = (sg_[0] + sg_[1]) / jnp.maximum(cg_[0] + cg_[1], 1.0)
    ah = (sh_[0] + sh_[1]) / jnp.maximum(ch_[0] + ch_[1], 1.0)
    z = pfx[...] + _dot(ag, wg[...]) + _dot(ah, wh[...]) + b1r[...][0:1, :]
    u = jnp.maximum(z, 0.0)
    fv = jnp.maximum(_dot(u, w2t[...]) + b2r[...][0:1, :], 0.0)
    out[...] = _dot(fv, wf[...])


def _fa_body(sa, ca, pfx, wg, b1r, w2t, b2r, out):
    sa_, ca_ = sa[...], ca[...]
    ag = (sa_[0] + sa_[1]) / jnp.maximum(ca_[0] + ca_[1], 1.0)
    z = pfx[...] + _dot(ag, wg[...]) + b1r[...][0:1, :]
    u = jnp.maximum(z, 0.0)
    out[...] = jnp.maximum(_dot(u, w2t[...]) + b2r[...][0:1, :], 0.0)


def _full(shape):
    return pl.BlockSpec(shape, lambda i: tuple(0 for _ in shape))


def _rows(bn, cols):
    return pl.BlockSpec((bn, cols), lambda i: (i, 0))


def _rows3(bn, cols):
    return pl.BlockSpec((NC, bn, cols), lambda i: (0, i, 0))


_n16 = jax.ShapeDtypeStruct((N, H), _f32)

_proj_call = pl.pallas_call(
    _proj_body,
    grid=(N // _BN,),
    in_specs=[_rows(_BN, D)] * 3 + [_full((D, 48)), _full((D, 16)),
                                    _full((D, 48))],
    out_specs=[_rows(_BN, H)] * 7,
    out_shape=tuple([_n16] * 7),
)

_edge_call = pl.pallas_call(
    _edge_body,
    grid=(EP // _BE,),
    in_specs=[_rows(_BE, H), _rows(_BE, H), _rows(_BE, 8),
              _full((8, H)), _full((H, H)), _full((8, H))],
    out_specs=_rows(_BE, H),
    out_shape=jax.ShapeDtypeStruct((EP, H), _f32),
)

_fv_call = pl.pallas_call(
    _fv_body,
    grid=(N // _BN,),
    in_specs=[_rows3(_BN, H)] * 4 + [_rows(_BN, H)] +
             [_full((H, H)), _full((H, H)), _full((8, H)), _full((H, H)),
              _full((8, H)), _full((H, H))],
    out_specs=_rows(_BN, H),
    out_shape=_n16,
)

_fa_call = pl.pallas_call(
    _fa_body,
    grid=(N // _BN,),
    in_specs=[_rows3(_BN, H)] * 2 + [_rows(_BN, H)] +
             [_full((H, H)), _full((8, H)), _full((H, H)), _full((8, H))],
    out_specs=_rows(_BN, H),
    out_shape=_n16,
)


def _pad_idx(idx, fill):
    p = jnp.pad(idx, (0, EP - E), constant_values=fill)
    return p.reshape(NW, K, CH)


def _rep8(b):
    return jnp.tile(b[None, :], (8, 1))


def kernel(x_c, x_v, x_a, edge_index_c2v, edge_index_a2v,
           edge_attr_c2v, edge_attr_a2v,
           gv_W1, gv_b1, gv_W2, gv_b2,
           hv_W1, hv_b1, hv_W2, hv_b2,
           fv_W1, fv_b1, fv_W2, fv_b2,
           ga_W1, ga_b1, ga_W2, ga_b2,
           fa_W1, fa_b1, fa_W2, fa_b2):
    c2v_s, c2v_t = edge_index_c2v[0], edge_index_c2v[1]
    a2v_s, a2v_t = edge_index_a2v[0], edge_index_a2v[1]

    wv = jnp.concatenate([gv_W1[:, :D], hv_W1[:, :D], fv_W1[:, :D]], 0).T
    wc = gv_W1[:, D:2 * D].T
    wa = jnp.concatenate([hv_W1[:, D:2 * D], ga_W1[:, :D], fa_W1[:, :D]], 0).T

    p_gvv, p_hvv, p_fvx, p_gvc, p_hva, p_gaa, p_fax = _proj_call(
        x_v, x_c, x_a, wv, wc, wa)

    def ea8(ea, wcols, b1):
        e8 = jnp.pad(jnp.concatenate([ea, jnp.ones((E, 1), _f32)], 1),
                     ((0, EP - E), (0, 3)))
        w8 = jnp.pad(jnp.concatenate([wcols, b1[:, None]], 1),
                     ((0, 0), (0, 3))).T
        return e8, w8

    eac8, wec = ea8(edge_attr_c2v, gv_W1[:, 2 * D:], gv_b1)
    eaa8, weh = ea8(edge_attr_a2v, hv_W1[:, 2 * D:], hv_b1)
    _, wea = ea8(edge_attr_a2v, ga_W1[:, D + H:], ga_b1)

    c2v_t_g = _pad_idx(c2v_t, 0)
    c2v_s_g = _pad_idx(c2v_s, 0)
    c2v_t_s = _pad_idx(c2v_t, DUMP)
    a2v_t_g = _pad_idx(a2v_t, 0)
    a2v_s_g = _pad_idx(a2v_s, 0)
    a2v_t_s = _pad_idx(a2v_t, DUMP)
    a2v_s_s = _pad_idx(a2v_s, DUMP)

    def as_edges(x):
        return x.reshape(EP, H)

    def as_chunks(x):
        return x.reshape(NW, SS, SCHS, H)

    zrows = jnp.zeros((RPS, H), _f32)

    g_t, g_s = _sc_gather(p_gvv, p_gvc, c2v_t_g, c2v_s_g)
    y_gv = _edge_call(as_edges(g_t), as_edges(g_s), eac8, wec, gv_W2.T,
                      _rep8(gv_b2))
    s_g, c_g = _sc_scatter(as_chunks(y_gv), c2v_t_s, zrows)

    h_t, h_s = _sc_gather(p_hvv, p_hva, a2v_t_g, a2v_s_g)
    y_hv = _edge_call(as_edges(h_t), as_edges(h_s), eaa8, weh, hv_W2.T,
                      _rep8(hv_b2))
    s_h, c_h = _sc_scatter(as_chunks(y_hv), a2v_t_s, zrows)

    p_gaf = _fv_call(s_g, c_g, s_h, c_h, p_fvx,
                     fv_W1[:, D:D + H].T, fv_W1[:, D + H:].T, _rep8(fv_b1),
                     fv_W2.T, _rep8(fv_b2), ga_W1[:, D:D + H].T)

    f_t, f_s = _sc_gather(p_gaf, p_gaa, a2v_t_g, a2v_s_g)
    y_ga = _edge_call(as_edges(f_t), as_edges(f_s), eaa8, wea, ga_W2.T,
                      _rep8(ga_b2))
    s_a, c_a = _sc_scatter(as_chunks(y_ga), a2v_s_s, zrows)

    return _fa_call(s_a, c_a, p_fax,
                    fa_W1[:, D:].T, _rep8(fa_b1), fa_W2.T, _rep8(fa_b2))


# speedup vs baseline: 4.0341x; 1.8249x over previous
import functools

import jax
import jax.numpy as jnp
from jax import lax
from jax.experimental import pallas as pl
from jax.experimental.pallas import tpu as pltpu
from jax.experimental.pallas import tpu_sc as plsc

N = 50000
E = 320000
D = 128
H = 16
NC = 2
NS = 16
NW = NC * NS
CH = 128
SCH = 1024
S = 10
EPW = S * SCH
EP = NW * EPW
EP8 = EP // 8
K = EPW // CH
NPAD = 50176
DUMP = NPAD - 128
RPS = NPAD // NS
SCHS = 512
SS = EPW // SCHS

_MESH = plsc.VectorSubcoreMesh(
    core_axis_name="c", subcore_axis_name="s", num_cores=NC, num_subcores=NS)

_SC_PARAMS = pltpu.CompilerParams(use_tc_tiling_on_sc=False)

_f32 = jnp.float32


@functools.partial(
    pl.kernel,
    out_type=(jax.ShapeDtypeStruct((NW, S, SCH, H), _f32),
              jax.ShapeDtypeStruct((NW, S, SCH, H), _f32)),
    mesh=_MESH,
    scratch_types=(
        pltpu.VMEM((K, CH), jnp.int32),
        pltpu.VMEM((K, CH), jnp.int32),
        pltpu.VMEM((2, SCH, H), _f32),
        pltpu.VMEM((2, SCH, H), _f32),
        pltpu.SemaphoreType.DMA,
        pltpu.SemaphoreType.DMA,
        pltpu.SemaphoreType.DMA,
        pltpu.SemaphoreType.DMA,
    ),
    compiler_params=_SC_PARAMS,
)
def _sc_gather(t1_flat, t2_flat, idxt_hbm, idxs_hbm, out1_hbm, out2_hbm,
               idxt_v, idxs_v, bufA, bufB, semg0, semg1, semw0, semw1):
    w = lax.axis_index("s") * NC + lax.axis_index("c")
    pltpu.sync_copy(idxt_hbm.at[w], idxt_v)
    pltpu.sync_copy(idxs_hbm.at[w], idxs_v)
    semg = (semg0, semg1)
    semw = (semw0, semw1)
    gh = {}
    wh = {}
    for s in range(S + 1):
        b = s % 2
        if s < S:
            for hnd in wh.pop(b, ()):
                hnd.wait()
            g = []
            for j in range(8):
                k = s * 8 + j
                dst = pl.ds(j * CH, CH)
                g.append(pltpu.async_copy(
                    t1_flat.at[idxt_v.at[k]], bufA.at[b, dst], semg[b]))
                g.append(pltpu.async_copy(
                    t2_flat.at[idxs_v.at[k]], bufB.at[b, dst], semg[b]))
            gh[b] = g
        if s >= 1:
            pb = (s - 1) % 2
            for hnd in gh.pop(pb):
                hnd.wait()
            wh[pb] = [
                pltpu.async_copy(bufA.at[pb], out1_hbm.at[w, s - 1], semw[pb]),
                pltpu.async_copy(bufB.at[pb], out2_hbm.at[w, s - 1], semw[pb]),
            ]
    for b in (0, 1):
        for hnd in wh.pop(b, ()):
            hnd.wait()


@functools.partial(
    pl.kernel,
    out_type=(jax.ShapeDtypeStruct((NC, NPAD, H), _f32),
              jax.ShapeDtypeStruct((NC, NPAD, H), _f32)),
    mesh=_MESH,
    scratch_types=(
        pltpu.VMEM_SHARED((NPAD, H), _f32),
        pltpu.VMEM_SHARED((NPAD, H), _f32),
        pltpu.VMEM((CH, H), _f32),
        pltpu.VMEM((K, CH), jnp.int32),
        pltpu.VMEM((2, SCHS, H), _f32),
        pltpu.SemaphoreType.DMA,
        pltpu.SemaphoreType.DMA,
        pltpu.SemaphoreType.DMA,
        pltpu.SemaphoreType.DMA,
        pltpu.SemaphoreType.DMA,
    ),
    compiler_params=_SC_PARAMS,
)
def _sc_scatter(y_hbm, idx_hbm, zeros_hbm, sum_hbm, cnt_hbm,
                acc_v, acc_c, obuf, idx_v, ybuf,
                semz, seml0, seml1, sems0, sems1):
    c = lax.axis_index("c")
    sid = lax.axis_index("s")
    w = sid * NC + c

    def fill_o(i, carry):
        obuf[i] = jnp.full((H,), 1.0, _f32)
        return carry

    lax.fori_loop(0, CH, fill_o, 0, unroll=8)

    row = pl.ds(sid * RPS, RPS)
    z0 = pltpu.async_copy(zeros_hbm, acc_v.at[row], semz)
    z1 = pltpu.async_copy(zeros_hbm, acc_c.at[row], semz)
    z0.wait()
    z1.wait()
    plsc.subcore_barrier()

    pltpu.sync_copy(idx_hbm.at[w], idx_v)
    seml = (seml0, seml1)
    sems = (sems0, sems1)
    lh = {}
    sh = {}
    for s in range(SS + 1):
        b = s % 2
        if s < SS:
            for hnd in sh.pop(b, ()):
                hnd.wait()
            lh[b] = pltpu.async_copy(y_hbm.at[w, s], ybuf.at[b], seml[b])
        if s >= 1:
            pb = (s - 1) % 2
            lh.pop(pb).wait()
            g = []
            for j in range(SCHS // CH):
                k = (s - 1) * (SCHS // CH) + j
                g.append(pltpu.async_copy(
                    ybuf.at[pb, pl.ds(j * CH, CH)], acc_v.at[idx_v.at[k]],
                    sems[pb], add=True))
                g.append(pltpu.async_copy(
                    obuf, acc_c.at[idx_v.at[k]], sems[pb], add=True))
            sh[pb] = g
    for b in (0, 1):
        for hnd in sh.pop(b, ()):
            hnd.wait()
    plsc.subcore_barrier()

    row = pl.ds(sid * RPS, RPS)
    pltpu.sync_copy(acc_v.at[row], sum_hbm.at[c, row])
    pltpu.sync_copy(acc_c.at[row], cnt_hbm.at[c, row])


_BN = 1000
_BP = 256

_J_GVV, _J_GVC, _J_HVV, _J_HVA, _J_GAA, _J_FVX, _J_FAX = range(7)


def _dot(a, b):
    return jnp.dot(a, b, preferred_element_type=_f32)


def _proj_body(xv, xc, xa, wv, wc, wa, out):
    pv = _dot(xv[...], wv[...])
    pc = _dot(xc[...], wc[...])
    pa = _dot(xa[...], wa[...])
    out[...] = jnp.concatenate(
        [pv[:, 0:16], pc, pv[:, 16:32], pa[:, 0:16], pa[:, 16:32],
         pv[:, 32:48], pa[:, 32:48], jnp.zeros_like(pc)], axis=1)


def _edge_body(gt, gs, eap, webd, w2bd, b2t, out):
    z = gt[...] + gs[...] + _dot(eap[...], webd[...])
    u = jnp.maximum(z, 0.0)
    out[...] = jnp.maximum(_dot(u, w2bd[...]) + b2t[...][0:1, :], 0.0)


def _fv_body(sg, cg, sh, ch, pall, wg, wh, b1r, w2t, b2r, wf, out):
    sg_, cg_, sh_, ch_ = sg[...], cg[...], sh[...], ch[...]
    ag = (sg_[0] + sg_[1]) / jnp.maximum(cg_[0] + cg_[1], 1.0)
    ah = (sh_[0] + sh_[1]) / jnp.maximum(ch_[0] + ch_[1], 1.0)
    pfx = pall[...][:, 16 * _J_FVX:16 * _J_FVX + 16]
    z = pfx + _dot(ag, wg[...]) + _dot(ah, wh[...]) + b1r[...][0:1, :]
    u = jnp.maximum(z, 0.0)
    fv = jnp.maximum(_dot(u, w2t[...]) + b2r[...][0:1, :], 0.0)
    pgaf = _dot(fv, wf[...])
    out[...] = jnp.pad(pgaf, ((0, 0), (0, D - H)))


def _fa_body(sa, ca, pall, wg, b1r, w2t, b2r, out):
    sa_, ca_ = sa[...], ca[...]
    ag = (sa_[0] + sa_[1]) / jnp.maximum(ca_[0] + ca_[1], 1.0)
    pfx = pall[...][:, 16 * _J_FAX:16 * _J_FAX + 16]
    z = pfx + _dot(ag, wg[...]) + b1r[...][0:1, :]
    u = jnp.maximum(z, 0.0)
    out[...] = jnp.maximum(_dot(u, w2t[...]) + b2r[...][0:1, :], 0.0)


def _full(shape):
    return pl.BlockSpec(shape, lambda i: tuple(0 for _ in shape))


def _rows(bn, cols):
    return pl.BlockSpec((bn, cols), lambda i: (i, 0))


def _rows3(bn, cols):
    return pl.BlockSpec((NC, bn, cols), lambda i: (0, i, 0))


_proj_call = pl.pallas_call(
    _proj_body,
    grid=(N // _BN,),
    in_specs=[_rows(_BN, D)] * 3 + [_full((D, 48)), _full((D, 16)),
                                    _full((D, 48))],
    out_specs=_rows(_BN, D),
    out_shape=jax.ShapeDtypeStruct((N, D), _f32),
)

_edge_call = pl.pallas_call(
    _edge_body,
    grid=(EP8 // _BP,),
    in_specs=[_rows(_BP, D), _rows(_BP, D), _rows(_BP, D),
              _full((D, D)), _full((D, D)), _full((8, D))],
    out_specs=_rows(_BP, D),
    out_shape=jax.ShapeDtypeStruct((EP8, D), _f32),
)

_fv_call = pl.pallas_call(
    _fv_body,
    grid=(N // _BN,),
    in_specs=[_rows3(_BN, H)] * 4 + [_rows(_BN, D)] +
             [_full((H, H)), _full((H, H)), _full((8, H)), _full((H, H)),
              _full((8, H)), _full((H, H))],
    out_specs=_rows(_BN, D),
    out_shape=jax.ShapeDtypeStruct((N, D), _f32),
)

_fa_call = pl.pallas_call(
    _fa_body,
    grid=(N // _BN,),
    in_specs=[_rows3(_BN, H)] * 2 + [_rows(_BN, D)] +
             [_full((H, H)), _full((8, H)), _full((H, H)), _full((8, H))],
    out_specs=_rows(_BN, H),
    out_shape=jax.ShapeDtypeStruct((N, H), _f32),
)


def _pad_idx(idx, fill):
    p = jnp.pad(idx, (0, EP - E), constant_values=fill)
    return p.reshape(NW, K, CH)


def _rep8(b):
    return jnp.tile(b[None, :], (8, 1))


def _bd(w16):
    return jnp.kron(jnp.eye(8, dtype=_f32), w16)


def kernel(x_c, x_v, x_a, edge_index_c2v, edge_index_a2v,
           edge_attr_c2v, edge_attr_a2v,
           gv_W1, gv_b1, gv_W2, gv_b2,
           hv_W1, hv_b1, hv_W2, hv_b2,
           fv_W1, fv_b1, fv_W2, fv_b2,
           ga_W1, ga_b1, ga_W2, ga_b2,
           fa_W1, fa_b1, fa_W2, fa_b2):
    c2v_s, c2v_t = edge_index_c2v[0], edge_index_c2v[1]
    a2v_s, a2v_t = edge_index_a2v[0], edge_index_a2v[1]

    wv = jnp.concatenate([gv_W1[:, :D], hv_W1[:, :D], fv_W1[:, :D]], 0).T
    wc = gv_W1[:, D:2 * D].T
    wa = jnp.concatenate([hv_W1[:, D:2 * D], ga_W1[:, :D], fa_W1[:, :D]], 0).T

    pall = _proj_call(x_v, x_c, x_a, wv, wc, wa)

    def eap_of(ea):
        e16 = jnp.concatenate(
            [ea, jnp.ones((E, 1), _f32), jnp.zeros((E, 11), _f32)], 1)
        return jnp.pad(e16.reshape(E // 8, D), ((0, EP8 - E // 8), (0, 0)))

    def we16(wcols, b1):
        return jnp.pad(jnp.concatenate([wcols, b1[:, None]], 1),
                       ((0, 0), (0, 11))).T

    eac = eap_of(edge_attr_c2v)
    eaa = eap_of(edge_attr_a2v)
    webd_gv = _bd(we16(gv_W1[:, 2 * D:], gv_b1))
    webd_hv = _bd(we16(hv_W1[:, 2 * D:], hv_b1))
    webd_ga = _bd(we16(ga_W1[:, D + H:], ga_b1))

    gi_gv_t = _pad_idx(8 * c2v_t + _J_GVV, 0)
    gi_gv_s = _pad_idx(8 * c2v_s + _J_GVC, 0)
    gi_hv_t = _pad_idx(8 * a2v_t + _J_HVV, 0)
    gi_hv_s = _pad_idx(8 * a2v_s + _J_HVA, 0)
    gi_ga_t = _pad_idx(8 * a2v_t, 0)
    gi_ga_s = _pad_idx(8 * a2v_s + _J_GAA, 0)
    si_gv = _pad_idx(c2v_t, DUMP)
    si_hv = _pad_idx(a2v_t, DUMP)
    si_ga = _pad_idx(a2v_s, DUMP)

    def as_packed(x):
        return x.reshape(EP8, D)

    def as_chunks(x):
        return x.reshape(NW, SS, SCHS, H)

    zrows = jnp.zeros((RPS, H), _f32)

    pall8 = pall.reshape(8 * N, H)
    g_t, g_s = _sc_gather(pall8, pall8, gi_gv_t, gi_gv_s)
    y_gv = _edge_call(as_packed(g_t), as_packed(g_s), eac, webd_gv,
                      _bd(gv_W2.T), _rep8(jnp.tile(gv_b2, 8)))
    s_g, c_g = _sc_scatter(as_chunks(y_gv), si_gv, zrows)

    h_t, h_s = _sc_gather(pall8, pall8, gi_hv_t, gi_hv_s)
    y_hv = _edge_call(as_packed(h_t), as_packed(h_s), eaa, webd_hv,
                      _bd(hv_W2.T), _rep8(jnp.tile(hv_b2, 8)))
    s_h, c_h = _sc_scatter(as_chunks(y_hv), si_hv, zrows)

    pallf = _fv_call(s_g, c_g, s_h, c_h, pall,
                     fv_W1[:, D:D + H].T, fv_W1[:, D + H:].T, _rep8(fv_b1),
                     fv_W2.T, _rep8(fv_b2), ga_W1[:, D:D + H].T)

    f_t, f_s = _sc_gather(pallf.reshape(8 * N, H), pall8, gi_ga_t, gi_ga_s)
    y_ga = _edge_call(as_packed(f_t), as_packed(f_s), eaa, webd_ga,
                      _bd(ga_W2.T), _rep8(jnp.tile(ga_b2, 8)))
    s_a, c_a = _sc_scatter(as_chunks(y_ga), si_ga, zrows)

    return _fa_call(s_a, c_a, pall,
                    fa_W1[:, D:].T, _rep8(fa_b1), fa_W2.T, _rep8(fa_b2))
